# baseline (speedup 1.0000x reference)
import jax
import jax.numpy as jnp
from jax import lax
from jax.experimental import pallas as pl
from jax.experimental.pallas import tpu as pltpu
from jax.experimental.pallas import tpu_sc as plsc

_NUM_EMBEDDINGS = 1024
_EMBEDDING_DIM = 64
_COMMITMENT_COST = 0.25
_ROWS_PER_STEP = 8


def _vq_tc_kernel(x_ref, emb_ref, idx2d_ref, idxf_ref, loss_ref):
    i = pl.program_id(0)
    blk = _ROWS_PER_STEP * x_ref.shape[1]
    x = x_ref[...].reshape(blk, _EMBEDDING_DIM)
    emb = emb_ref[...]
    x2 = jnp.sum(x ** 2, axis=1, keepdims=True)
    e2 = jnp.sum(emb ** 2, axis=1)
    mm = jax.lax.dot_general(
        x * -2.0, emb, (((1,), (1,)), ((), ())),
        preferred_element_type=jnp.float32,
    )
    scores = (x2 + e2[None, :]) + mm

    idx = jnp.argmin(scores, axis=1).astype(jnp.int32)
    idx2d_ref[...] = idx.reshape(_ROWS_PER_STEP, x_ref.shape[1])
    idxf_ref[pl.ds(i * blk, blk)] = idx

    part = jnp.sum(jnp.min(scores, axis=1)).reshape(1, 1)

    @pl.when(i == 0)
    def _():
        loss_ref[...] = part

    @pl.when(i != 0)
    def _():
        loss_ref[...] += part


def _sc_gather(emb_hbm, idx_hbm, out_hbm, idx_v, rows_v, sem):
    info = plsc.get_sparse_core_info()
    nw = info.num_cores * info.num_subcores
    b_per_w = out_hbm.shape[0] // nw
    wid = lax.axis_index("s") * info.num_cores + lax.axis_index("c")
    base = wid * b_per_w
    pltpu.sync_copy(idx_hbm.at[pl.ds(base, b_per_w)], idx_v)
    pltpu.async_copy(emb_hbm.at[idx_v], rows_v, sem).wait()
    pltpu.sync_copy(rows_v, out_hbm.at[pl.ds(base, b_per_w)])


def kernel(inputs, embedding_weight):
    nmaj, nmin, _ = inputs.shape
    n = nmaj * nmin
    nb = nmaj // _ROWS_PER_STEP
    blk = _ROWS_PER_STEP * nmin
    idx2d, idxf, loss_acc = pl.pallas_call(
        _vq_tc_kernel,
        grid=(nb,),
        in_specs=[
            pl.BlockSpec((_ROWS_PER_STEP, nmin, _EMBEDDING_DIM),
                         lambda i: (i, 0, 0)),
            pl.BlockSpec((_NUM_EMBEDDINGS, _EMBEDDING_DIM), lambda i: (0, 0)),
        ],
        out_specs=[
            pl.BlockSpec((_ROWS_PER_STEP, nmin), lambda i: (i, 0)),
            pl.BlockSpec((n,), lambda i: (0,)),
            pl.BlockSpec((1, 1), lambda i: (0, 0)),
        ],
        out_shape=[
            jax.ShapeDtypeStruct((nmaj, nmin), jnp.int32),
            jax.ShapeDtypeStruct((n,), jnp.int32),
            jax.ShapeDtypeStruct((1, 1), jnp.float32),
        ],
    )(inputs, embedding_weight)

    info = plsc.get_sparse_core_info()
    nw = info.num_cores * info.num_subcores
    b_per_w = n // nw
    gather = pl.kernel(
        _sc_gather,
        mesh=plsc.VectorSubcoreMesh(core_axis_name="c", subcore_axis_name="s"),
        compiler_params=pltpu.CompilerParams(use_tc_tiling_on_sc=False),
        out_type=jax.ShapeDtypeStruct((n, _EMBEDDING_DIM), jnp.float32),
        scratch_types=[
            pltpu.VMEM((b_per_w,), jnp.int32),
            pltpu.VMEM((b_per_w, _EMBEDDING_DIM), jnp.float32),
            pltpu.SemaphoreType.DMA,
        ],
    )
    q = gather(embedding_weight, idxf).reshape(inputs.shape)
    loss = _COMMITMENT_COST * loss_acc[0, 0] / inputs.size
    return (q, loss, idx2d)


# speedup vs baseline: 1.0294x; 1.0294x over previous
import functools

import jax
import jax.numpy as jnp
from jax import lax
from jax.experimental import pallas as pl
from jax.experimental.pallas import tpu as pltpu
from jax.experimental.pallas import tpu_sc as plsc

_NUM_EMBEDDINGS = 1024
_EMBEDDING_DIM = 64
_COMMITMENT_COST = 0.25
_ROWS_PER_STEP = 8


def _vq_tc_kernel(x_ref, emb_ref, idx_ref, loss_ref):
    i = pl.program_id(0)
    blk = _ROWS_PER_STEP * x_ref.shape[1]
    x = x_ref[...].reshape(blk, _EMBEDDING_DIM)
    emb = emb_ref[...]
    x2 = jnp.sum(x ** 2, axis=1, keepdims=True)
    e2 = jnp.sum(emb ** 2, axis=1)
    mm = jax.lax.dot_general(
        x * -2.0, emb, (((1,), (1,)), ((), ())),
        preferred_element_type=jnp.float32,
    )
    scores = (x2 + e2[None, :]) + mm

    idx = jnp.argmin(scores, axis=1).astype(jnp.int32)
    idx_ref[...] = idx.reshape(_ROWS_PER_STEP, x_ref.shape[1])

    code_iota = jax.lax.broadcasted_iota(jnp.int32, scores.shape, 1)
    onehot = (code_iota == idx[:, None]).astype(jnp.float32)
    q = jax.lax.dot_general(
        onehot, emb, (((1,), (0,)), ((), ())),
        preferred_element_type=jnp.float32,
    )

    d = q - x
    part = jnp.sum(d * d).reshape(1, 1)

    @pl.when(i == 0)
    def _():
        loss_ref[...] = part

    @pl.when(i != 0)
    def _():
        loss_ref[...] += part


def _make_sc_gather(batch, dim):
    info = plsc.get_sparse_core_info()
    num_workers = info.num_cores * info.num_subcores
    b_per_w = batch // num_workers
    mesh = plsc.VectorSubcoreMesh(core_axis_name="c", subcore_axis_name="s")

    chunk = b_per_w // 2

    @functools.partial(
        pl.kernel, mesh=mesh,
        out_type=jax.ShapeDtypeStruct((batch, 128), jnp.float32),
        scratch_types=[
            pltpu.VMEM((chunk,), jnp.int32),
            pltpu.VMEM((chunk, 128), jnp.float32),
            pltpu.SemaphoreType.DMA,
        ],
    )
    def sc_gather(table_hbm, idx_hbm, out_hbm, idx_v, rows_v, sem):
        wid = lax.axis_index("s") * info.num_cores + lax.axis_index("c")
        for c in range(2):
            base = wid * b_per_w + c * chunk
            pltpu.sync_copy(idx_hbm.at[pl.ds(base, chunk)], idx_v)
            pltpu.async_copy(table_hbm.at[idx_v], rows_v, sem).wait()
            pltpu.sync_copy(rows_v, out_hbm.at[pl.ds(base, chunk)])

    return sc_gather


def kernel(inputs, embedding_weight):
    nmaj, nmin, _ = inputs.shape
    nb = nmaj // _ROWS_PER_STEP
    idx, loss_acc = pl.pallas_call(
        _vq_tc_kernel,
        grid=(nb,),
        in_specs=[
            pl.BlockSpec((_ROWS_PER_STEP, nmin, _EMBEDDING_DIM),
                         lambda i: (i, 0, 0)),
            pl.BlockSpec((_NUM_EMBEDDINGS, _EMBEDDING_DIM), lambda i: (0, 0)),
        ],
        out_specs=[
            pl.BlockSpec((_ROWS_PER_STEP, nmin), lambda i: (i, 0)),
            pl.BlockSpec((1, 1), lambda i: (0, 0)),
        ],
        out_shape=[
            jax.ShapeDtypeStruct((nmaj, nmin), jnp.int32),
            jax.ShapeDtypeStruct((1, 1), jnp.float32),
        ],
    )(inputs, embedding_weight)

    batch = nmaj * nmin
    table_padded = jnp.pad(embedding_weight, ((0, 0), (0, 64)))
    q = _make_sc_gather(batch, _EMBEDDING_DIM)(
        table_padded, idx.reshape(batch))
    q = q[:, :_EMBEDDING_DIM].reshape(nmaj, nmin, _EMBEDDING_DIM)

    loss = _COMMITMENT_COST * loss_acc[0, 0] / inputs.size
    return (q, loss, idx)
